# baseline (speedup 1.0000x reference)
import functools

import jax
import jax.numpy as jnp
from jax import lax
from jax.experimental import pallas as pl
from jax.experimental.pallas import tpu as pltpu
from jax.experimental.pallas import tpu_sc as plsc

C = 768
R = 50176
K = 64
NW = 32
RPW = R // NW
CH = 56
NCH = RPW // CH
NG = C // 16
RCHUNK = 128

_MESH = plsc.VectorSubcoreMesh(core_axis_name="c", subcore_axis_name="s")


@functools.partial(
    pl.kernel,
    mesh=_MESH,
    out_type=jax.ShapeDtypeStruct((NW, C), jnp.float32),
    scratch_types=[
        pltpu.VMEM((2, CH, C), jnp.float32),
        pltpu.VMEM((C,), jnp.float32),
        pltpu.SemaphoreType.DMA((2,)),
    ],
)
def _sc_sums(x_hbm, out_hbm, buf, acc, sems):
    cid = lax.axis_index("c")
    sid = lax.axis_index("s")
    wid = sid * 2 + cid
    base = wid * RPW

    zero = jnp.zeros((16,), jnp.float32)
    for g in range(NG):
        acc[pl.ds(g * 16, 16)] = zero

    def copy(k, slot):
        return pltpu.make_async_copy(
            x_hbm.at[pl.ds(base + k * CH, CH)], buf.at[slot], sems.at[slot]
        )

    copy(0, 0).start()
    copy(1, 1).start()

    def pair(k2, _):
        for b in range(2):
            k = k2 * 2 + b
            copy(k, b).wait()

            def block(r, _, b=b):
                for g in range(NG):
                    s = buf[b, r * 8, pl.ds(g * 16, 16)]
                    for i in range(1, 8):
                        s = s + buf[b, r * 8 + i, pl.ds(g * 16, 16)]
                    acc[pl.ds(g * 16, 16)] += s
                return 0

            lax.fori_loop(0, CH // 8, block, 0)

            @pl.when(k + 2 < NCH)
            def _refill(k=k, b=b):
                copy(k + 2, b).start()
        return 0

    lax.fori_loop(0, NCH // 2, pair, 0)
    pltpu.sync_copy(acc, out_hbm.at[wid])


def _topk_body(p_ref, idx_ref):
    totals = jnp.sum(p_ref[...], axis=0)
    vj = totals[None, :]
    jj = lax.broadcasted_iota(jnp.int32, (RCHUNK, C), 1)
    ranks = []
    for c in range(C // RCHUNK):
        vi = totals[c * RCHUNK:(c + 1) * RCHUNK][:, None]
        ii = lax.broadcasted_iota(jnp.int32, (RCHUNK, C), 0) + c * RCHUNK
        beats = (vj > vi) | ((vj == vi) & (jj < ii))
        ranks.append(jnp.sum(beats.astype(jnp.int32), axis=1))
    rank = jnp.concatenate(ranks)
    tsel = lax.broadcasted_iota(jnp.int32, (K, C), 0)
    chan = lax.broadcasted_iota(jnp.int32, (K, C), 1)
    onehot = (rank[None, :] == tsel)
    idx_ref[...] = jnp.sum(jnp.where(onehot, chan, 0), axis=1)


def kernel(input):
    x = jnp.transpose(input, (0, 2, 3, 1)).reshape(R, C)
    partials = _sc_sums(x)
    return pl.pallas_call(
        _topk_body,
        out_shape=jax.ShapeDtypeStruct((K,), jnp.int32),
    )(partials)


# speedup vs baseline: 2.4316x; 2.4316x over previous
import functools

import jax
import jax.numpy as jnp
from jax import lax
from jax.experimental import pallas as pl
from jax.experimental.pallas import tpu as pltpu
from jax.experimental.pallas import tpu_sc as plsc

C = 768
R = 50176
K = 64
RCHUNK = 128

RB = 1792
NCHUNK = 22
R_TC = RB * NCHUNK
NBUF = 6

NW = 32
R_SC = R - R_TC
RPW = R_SC // NW
CH = 56
NCH = RPW // CH
NG = C // 16


def _tc_sum_body(x_hbm, sums_ref, bufs, sems, acc_ref):
    j = pl.program_id(0)

    def start(chunk, slot):
        pltpu.make_async_copy(
            x_hbm.at[pl.ds(chunk * RB, RB)], bufs.at[slot], sems.at[slot]
        ).start()

    @pl.when(j == 0)
    def _prime():
        acc_ref[...] = jnp.zeros_like(acc_ref)
        for b in range(NBUF):
            start(b, b)

    slot = lax.rem(j, NBUF)
    pltpu.make_async_copy(
        x_hbm.at[pl.ds(j * RB, RB)], bufs.at[slot], sems.at[slot]
    ).wait()
    acc_ref[...] += jnp.sum(bufs[slot].reshape(RB // 8, 8, C), axis=0)

    @pl.when(j + NBUF < NCHUNK)
    def _refill():
        start(j + NBUF, slot)

    @pl.when(j == NCHUNK - 1)
    def _finish():
        sums_ref[...] = jnp.sum(acc_ref[...], axis=0)


_MESH = plsc.VectorSubcoreMesh(core_axis_name="c", subcore_axis_name="s")


@functools.partial(
    pl.kernel,
    mesh=_MESH,
    out_type=jax.ShapeDtypeStruct((NW, C), jnp.float32),
    scratch_types=[
        pltpu.VMEM((2, CH, C), jnp.float32),
        pltpu.VMEM((C,), jnp.float32),
        pltpu.SemaphoreType.DMA((2,)),
    ],
)
def _sc_sums(x_hbm, out_hbm, buf, acc, sems):
    cid = lax.axis_index("c")
    sid = lax.axis_index("s")
    wid = sid * 2 + cid
    base = R_TC + wid * RPW

    zero = jnp.zeros((16,), jnp.float32)
    for g in range(NG):
        acc[pl.ds(g * 16, 16)] = zero

    def copy(k, slot):
        return pltpu.make_async_copy(
            x_hbm.at[pl.ds(base + k * CH, CH)], buf.at[slot], sems.at[slot]
        )

    copy(0, 0).start()
    copy(1, 1).start()

    def pair(k2, _):
        for b in range(2):
            k = k2 * 2 + b
            copy(k, b).wait()

            def block(r, _, b=b):
                for g in range(NG):
                    s = buf[b, r * 8, pl.ds(g * 16, 16)]
                    for i in range(1, 8):
                        s = s + buf[b, r * 8 + i, pl.ds(g * 16, 16)]
                    acc[pl.ds(g * 16, 16)] += s
                return 0

            lax.fori_loop(0, CH // 8, block, 0)

            @pl.when(k + 2 < NCH)
            def _refill(k=k, b=b):
                copy(k + 2, b).start()
        return 0

    lax.fori_loop(0, NCH // 2, pair, 0)
    pltpu.sync_copy(acc, out_hbm.at[wid])


def _topk_body(tc_ref, p_ref, idx_ref):
    totals = tc_ref[...] + jnp.sum(p_ref[...], axis=0)
    vj = totals[None, :]
    jj = lax.broadcasted_iota(jnp.int32, (RCHUNK, C), 1)
    ranks = []
    for c in range(C // RCHUNK):
        vi = totals[c * RCHUNK:(c + 1) * RCHUNK][:, None]
        ii = lax.broadcasted_iota(jnp.int32, (RCHUNK, C), 0) + c * RCHUNK
        beats = (vj > vi) | ((vj == vi) & (jj < ii))
        ranks.append(jnp.sum(beats.astype(jnp.int32), axis=1))
    rank = jnp.concatenate(ranks)
    tsel = lax.broadcasted_iota(jnp.int32, (K, C), 0)
    chan = lax.broadcasted_iota(jnp.int32, (K, C), 1)
    onehot = (rank[None, :] == tsel)
    idx_ref[...] = jnp.sum(jnp.where(onehot, chan, 0), axis=1)


def kernel(input):
    x = jnp.transpose(input, (0, 2, 3, 1)).reshape(R, C)
    sc_partials = _sc_sums(x)
    tc_sums = pl.pallas_call(
        _tc_sum_body,
        grid=(NCHUNK,),
        in_specs=[pl.BlockSpec(memory_space=pl.ANY)],
        out_specs=pl.BlockSpec((C,), lambda j: (0,)),
        out_shape=jax.ShapeDtypeStruct((C,), jnp.float32),
        scratch_shapes=[
            pltpu.VMEM((NBUF, RB, C), jnp.float32),
            pltpu.SemaphoreType.DMA((NBUF,)),
            pltpu.VMEM((8, C), jnp.float32),
        ],
    )(x)
    return pl.pallas_call(
        _topk_body,
        out_shape=jax.ShapeDtypeStruct((K,), jnp.int32),
    )(tc_sums, sc_partials)


# speedup vs baseline: 3.3331x; 1.3708x over previous
import jax
import jax.numpy as jnp
from jax import lax
from jax.experimental import pallas as pl
from jax.experimental.pallas import tpu as pltpu

C = 768
R = 50176
K = 64
RB = 1792
NCHUNK = R // RB
NBUF = 6
RCHUNK = 128


def _body(x_hbm, idx_ref, bufs, sems, acc_ref):
    j = pl.program_id(0)

    def start(chunk, slot):
        pltpu.make_async_copy(
            x_hbm.at[pl.ds(chunk * RB, RB)], bufs.at[slot], sems.at[slot]
        ).start()

    @pl.when(j == 0)
    def _prime():
        acc_ref[...] = jnp.zeros_like(acc_ref)
        for b in range(NBUF):
            start(b, b)

    slot = lax.rem(j, NBUF)
    pltpu.make_async_copy(
        x_hbm.at[pl.ds(j * RB, RB)], bufs.at[slot], sems.at[slot]
    ).wait()
    acc_ref[...] += jnp.sum(bufs[slot].reshape(RB // 8, 8, C), axis=0)

    @pl.when(j + NBUF < NCHUNK)
    def _refill():
        start(j + NBUF, slot)

    @pl.when(j == NCHUNK - 1)
    def _finish():
        totals = jnp.sum(acc_ref[...], axis=0)
        vj = totals[None, :]
        jj = lax.broadcasted_iota(jnp.int32, (RCHUNK, C), 1)
        ranks = []
        for c in range(C // RCHUNK):
            vi = totals[c * RCHUNK:(c + 1) * RCHUNK][:, None]
            ii = lax.broadcasted_iota(jnp.int32, (RCHUNK, C), 0) + c * RCHUNK
            beats = (vj > vi) | ((vj == vi) & (jj < ii))
            ranks.append(jnp.sum(beats.astype(jnp.int32), axis=1))
        rank = jnp.concatenate(ranks)
        tsel = lax.broadcasted_iota(jnp.int32, (K, C), 0)
        chan = lax.broadcasted_iota(jnp.int32, (K, C), 1)
        onehot = (rank[None, :] == tsel)
        idx_ref[...] = jnp.sum(jnp.where(onehot, chan, 0), axis=1)


def kernel(input):
    x = jnp.transpose(input, (0, 2, 3, 1)).reshape(R, C)
    return pl.pallas_call(
        _body,
        grid=(NCHUNK,),
        in_specs=[pl.BlockSpec(memory_space=pl.ANY)],
        out_specs=pl.BlockSpec((K,), lambda j: (0,)),
        out_shape=jax.ShapeDtypeStruct((K,), jnp.int32),
        scratch_shapes=[
            pltpu.VMEM((NBUF, RB, C), jnp.float32),
            pltpu.SemaphoreType.DMA((NBUF,)),
            pltpu.VMEM((8, C), jnp.float32),
        ],
    )(x)
